# Initial kernel scaffold; baseline (speedup 1.0000x reference)
import jax
import jax.numpy as jnp
from jax.experimental import pallas as pl


def kernel(x, emb_table, pos_table):
    raise NotImplementedError("write your pallas kernel here")


# speedup vs baseline: 2.5950x; 2.5950x over previous
import functools

import jax
import jax.numpy as jnp
from jax import lax
from jax.experimental import pallas as pl
from jax.experimental.pallas import tpu as pltpu
from jax.experimental.pallas import tpu_sc as plsc

_L = 16


def _emb_body(L, H, seqs_per_worker, num_cores,
              x_hbm, emb_hbm, pos_hbm, out_hbm,
              idx_v, rows_v, pos_v, sem):
  wid = lax.axis_index("s") * num_cores + lax.axis_index("c")

  pltpu.sync_copy(pos_hbm, pos_v)

  def one_seq(s, _):
    seq = wid * seqs_per_worker + s
    base = seq * L
    pltpu.sync_copy(x_hbm.at[pl.ds(base, L)], idx_v)
    pltpu.async_copy(emb_hbm.at[idx_v], rows_v, sem).wait()

    def add_row(j, _):
      for h in range(H // _L):
        sl = pl.ds(h * _L, _L)
        rows_v[j, sl] = rows_v[j, sl] + pos_v[j, sl]
      return 0

    lax.fori_loop(0, L, add_row, 0)
    pltpu.sync_copy(rows_v, out_hbm.at[pl.ds(base, L)])
    return 0

  lax.fori_loop(0, seqs_per_worker, one_seq, 0)


def kernel(x, emb_table, pos_table):
  B, L = x.shape
  V, H = emb_table.shape
  info = plsc.get_sparse_core_info()
  nw = info.num_cores * info.num_subcores
  seqs_per_worker = B // nw

  mesh = plsc.VectorSubcoreMesh(core_axis_name="c", subcore_axis_name="s")
  body = functools.partial(_emb_body, L, H, seqs_per_worker, info.num_cores)
  run = pl.kernel(
      body,
      out_type=jax.ShapeDtypeStruct((B * L, H), jnp.float32),
      mesh=mesh,
      scratch_types=[
          pltpu.VMEM((L,), jnp.int32),
          pltpu.VMEM((L, H), jnp.float32),
          pltpu.VMEM((L, H), jnp.float32),
          pltpu.SemaphoreType.DMA,
      ],
      compiler_params=pltpu.CompilerParams(use_tc_tiling_on_sc=False),
  )
  out = run(x.reshape(-1).astype(jnp.int32), emb_table, pos_table)
  return out.reshape(B, L, H)


# speedup vs baseline: 3.1205x; 1.2025x over previous
import functools

import jax
import jax.numpy as jnp
from jax import lax
from jax.experimental import pallas as pl
from jax.experimental.pallas import tpu as pltpu
from jax.experimental.pallas import tpu_sc as plsc

_L = 16
_SEQS_PER_CHUNK = 2


def _emb_body(L, H, seqs_per_worker, num_cores,
              x_hbm, emb_hbm, pos_hbm, out_hbm,
              idx_v, pos_v, rows0, rows1, sem_g0, sem_g1, sem_o0, sem_o1):
  wid = lax.axis_index("s") * num_cores + lax.axis_index("c")
  rows_per_worker = seqs_per_worker * L
  chunk_rows = _SEQS_PER_CHUNK * L
  num_chunks = seqs_per_worker // _SEQS_PER_CHUNK
  base = wid * rows_per_worker

  bufs = (rows0, rows1)
  gsems = (sem_g0, sem_g1)
  osems = (sem_o0, sem_o1)

  pltpu.sync_copy(x_hbm.at[pl.ds(base, rows_per_worker)], idx_v)
  pltpu.sync_copy(pos_hbm, pos_v)

  def start_gather(c):
    b = c % 2
    return pltpu.async_copy(
        emb_hbm.at[idx_v.at[pl.ds(c * chunk_rows, chunk_rows)]],
        bufs[b], gsems[b])

  def add_pos(buf):
    def add_row(j, _):
      for s in range(_SEQS_PER_CHUNK):
        for h in range(H // _L):
          sl = pl.ds(h * _L, _L)
          plsc.addupdate(buf.at[s * L + j, sl], pos_v[j, sl])
      return 0
    lax.fori_loop(0, L, add_row, 0)

  def start_out(c):
    b = c % 2
    return pltpu.async_copy(
        bufs[b], out_hbm.at[pl.ds(base + c * chunk_rows, chunk_rows)],
        osems[b])

  gathers = {0: start_gather(0)}
  outs = {}
  for c in range(num_chunks):
    b = c % 2
    if c + 1 < num_chunks:
      if c - 1 >= 0:
        outs[c - 1].wait()
      gathers[c + 1] = start_gather(c + 1)
    gathers[c].wait()
    add_pos(bufs[b])
    outs[c] = start_out(c)
  outs[num_chunks - 2].wait()
  outs[num_chunks - 1].wait()


def kernel(x, emb_table, pos_table):
  B, L = x.shape
  V, H = emb_table.shape
  info = plsc.get_sparse_core_info()
  nw = info.num_cores * info.num_subcores
  seqs_per_worker = B // nw

  mesh = plsc.VectorSubcoreMesh(core_axis_name="c", subcore_axis_name="s")
  body = functools.partial(_emb_body, L, H, seqs_per_worker, info.num_cores)
  run = pl.kernel(
      body,
      out_type=jax.ShapeDtypeStruct((B * L, H), jnp.float32),
      mesh=mesh,
      scratch_types=[
          pltpu.VMEM((seqs_per_worker * L,), jnp.int32),
          pltpu.VMEM((L, H), jnp.float32),
          pltpu.VMEM((_SEQS_PER_CHUNK * L, H), jnp.float32),
          pltpu.VMEM((_SEQS_PER_CHUNK * L, H), jnp.float32),
          pltpu.SemaphoreType.DMA,
          pltpu.SemaphoreType.DMA,
          pltpu.SemaphoreType.DMA,
          pltpu.SemaphoreType.DMA,
      ],
      compiler_params=pltpu.CompilerParams(use_tc_tiling_on_sc=False),
  )
  out = run(x.reshape(-1).astype(jnp.int32), emb_table, pos_table)
  return out.reshape(B, L, H)
